# baseline (device time: 25854 ns/iter reference)
import jax
import jax.numpy as jnp
from jax import lax
from jax.experimental import pallas as pl
from jax.experimental.pallas import tpu as pltpu

NJ = 4


def kernel(Q, K, V):
    b, s, h, d = Q.shape
    bs, hd = b * s, h * d
    rows = 2 * bs
    ch = rows // (2 * NJ)
    scale = d ** -0.5

    def body(q_ref, k_ref, v_ref, out_ref, kv_send, kv_rem,
             x_send_sems, fwd_send_sems, recv_sems):
        my_x = lax.axis_index("x")
        my_y = lax.axis_index("y")
        my_z = lax.axis_index("z")
        p = my_y % 2
        xnbr = (1 - my_x, my_y, my_z)
        ynbr = (my_x, my_y + 1 - 2 * p, my_z)

        barrier_sem = pltpu.get_barrier_semaphore()
        for nb in (xnbr, ynbr):
            pl.semaphore_signal(
                barrier_sem, inc=1, device_id=nb,
                device_id_type=pl.DeviceIdType.MESH,
            )
        pl.semaphore_wait(barrier_sem, 2)

        kv_send[0:bs, :] = k_ref[...].astype(jnp.bfloat16)
        kv_send[bs:rows, :] = v_ref[...].astype(jnp.bfloat16)

        x_rdmas = []
        for j in range(NJ):
            row0 = (2 * j + p) * ch
            rdma = pltpu.make_async_remote_copy(
                src_ref=kv_send.at[pl.ds(row0, ch)],
                dst_ref=kv_rem.at[pl.ds(row0, ch)],
                send_sem=x_send_sems.at[j],
                recv_sem=recv_sems.at[j],
                device_id=xnbr,
                device_id_type=pl.DeviceIdType.MESH,
            )
            rdma.start()
            x_rdmas.append(rdma)

        qbs, l0s, o0s = [], [], []
        for bi in range(b):
            r0 = bi * s
            for hi in range(h):
                c0 = hi * d
                qb = (q_ref[r0:r0 + s, c0:c0 + d] * scale).astype(
                    jnp.bfloat16)
                kb = kv_send[r0:r0 + s, c0:c0 + d]
                vb = kv_send[bs + r0:bs + r0 + s, c0:c0 + d]
                s0 = lax.dot_general(
                    qb, kb, (((1,), (1,)), ((), ())),
                    preferred_element_type=jnp.float32,
                )
                p0 = jnp.exp(s0)
                l0 = jnp.sum(p0, axis=1, keepdims=True)
                o0 = jnp.dot(
                    p0.astype(jnp.bfloat16), vb,
                    preferred_element_type=jnp.float32,
                )
                qbs.append(qb)
                l0s.append(l0)
                o0s.append(o0)

        fwds = []
        for j in range(NJ):
            row0 = (2 * j + p) * ch
            x_rdmas[j].wait()
            fwd = pltpu.make_async_remote_copy(
                src_ref=kv_rem.at[pl.ds(row0, ch)],
                dst_ref=kv_rem.at[pl.ds(row0, ch)],
                send_sem=fwd_send_sems.at[j],
                recv_sem=recv_sems.at[NJ + j],
                device_id=ynbr,
                device_id_type=pl.DeviceIdType.MESH,
            )
            fwd.start()
            fwds.append(fwd)
        for j in range(NJ):
            row0 = (2 * j + 1 - p) * ch
            recv = pltpu.make_async_remote_copy(
                src_ref=kv_rem.at[pl.ds(row0, ch)],
                dst_ref=kv_rem.at[pl.ds(row0, ch)],
                send_sem=fwd_send_sems.at[j],
                recv_sem=recv_sems.at[NJ + j],
                device_id=ynbr,
                device_id_type=pl.DeviceIdType.MESH,
            )
            recv.wait_recv()

        for bi in range(b):
            r0 = bi * s
            for hi in range(h):
                c0 = hi * d
                i = bi * h + hi
                qb, l0, o0 = qbs[i], l0s[i], o0s[i]
                kb = kv_rem[r0:r0 + s, c0:c0 + d]
                vb = kv_rem[bs + r0:bs + r0 + s, c0:c0 + d]
                s1 = lax.dot_general(
                    qb, kb, (((1,), (1,)), ((), ())),
                    preferred_element_type=jnp.float32,
                )
                p1 = jnp.exp(s1)
                l1 = jnp.sum(p1, axis=1, keepdims=True)
                o1 = jnp.dot(
                    p1.astype(jnp.bfloat16), vb,
                    preferred_element_type=jnp.float32,
                )
                out_ref[r0:r0 + s, c0:c0 + d] = (o0 + o1) / (l0 + l1)

        for j in range(NJ):
            fwds[j].wait_send()

    out2 = pl.pallas_call(
        body,
        out_shape=jax.ShapeDtypeStruct((bs, hd), jnp.float32),
        in_specs=[
            pl.BlockSpec(memory_space=pltpu.VMEM),
            pl.BlockSpec(memory_space=pltpu.VMEM),
            pl.BlockSpec(memory_space=pltpu.VMEM),
        ],
        out_specs=pl.BlockSpec(memory_space=pltpu.VMEM),
        scratch_shapes=[
            pltpu.VMEM((rows, hd), jnp.bfloat16),
            pltpu.VMEM((rows, hd), jnp.bfloat16),
            pltpu.SemaphoreType.DMA((NJ,)),
            pltpu.SemaphoreType.DMA((NJ,)),
            pltpu.SemaphoreType.DMA((2 * NJ,)),
        ],
        compiler_params=pltpu.CompilerParams(collective_id=0),
    )(Q.reshape(bs, hd), K.reshape(bs, hd), V.reshape(bs, hd))
    return out2.reshape(b, s, h, d)
